# Initial kernel scaffold; baseline (speedup 1.0000x reference)
import functools

import jax
import jax.numpy as jnp
from jax import lax
from jax.experimental import pallas as pl
from jax.experimental.pallas import tpu as pltpu
from jax.experimental.pallas import tpu_sc as plsc

N = 10000
D = 128
E = 320000

N_PAD = 10240
NC, NS = 2, 16
W = NC * NS
CH = 128
EPW = 10240
NCH = EPW // CH
E_PAD = W * EPW
ROWS_PER_TILE = N_PAD // NS

_mesh = plsc.VectorSubcoreMesh(core_axis_name="c", subcore_axis_name="s")


def _sc_body(with_cnt, *refs):
    if with_cnt:
        (g_hbm, src_hbm, dst_hbm, zeros_hbm, zeros16_hbm, ones_hbm,
         out_hbm, cnt_hbm,
         src_v, dst_v, rows_v, ones_v, acc_sh, cnt_sh, sem) = refs
    else:
        (g_hbm, src_hbm, dst_hbm, zeros_hbm,
         out_hbm,
         src_v, dst_v, rows_v, acc_sh, sem) = refs

    cc = lax.axis_index("c")
    ss = lax.axis_index("s")
    wid = cc * NS + ss
    r0 = ss * ROWS_PER_TILE

    pltpu.sync_copy(zeros_hbm.at[pl.ds(r0, ROWS_PER_TILE)],
                    acc_sh.at[pl.ds(r0, ROWS_PER_TILE)])
    pltpu.sync_copy(src_hbm.at[wid], src_v)
    pltpu.sync_copy(dst_hbm.at[wid], dst_v)
    if with_cnt:
        pltpu.sync_copy(zeros16_hbm.at[pl.ds(r0, ROWS_PER_TILE)],
                        cnt_sh.at[pl.ds(r0, ROWS_PER_TILE)])
        pltpu.sync_copy(ones_hbm, ones_v)
    plsc.subcore_barrier()

    def chunk(i, carry):
        sidx = src_v.at[i]
        didx = dst_v.at[i]
        pltpu.async_copy(g_hbm.at[sidx], rows_v, sem).wait()
        pltpu.sync_copy(rows_v, acc_sh.at[didx], add=True)
        if with_cnt:
            pltpu.sync_copy(ones_v, cnt_sh.at[didx], add=True)
        return carry

    lax.fori_loop(0, NCH, chunk, 0)
    plsc.subcore_barrier()

    pltpu.sync_copy(acc_sh.at[pl.ds(r0, ROWS_PER_TILE)],
                    out_hbm.at[cc, pl.ds(r0, ROWS_PER_TILE)])
    if with_cnt:
        pltpu.sync_copy(cnt_sh.at[pl.ds(r0, ROWS_PER_TILE)],
                        cnt_hbm.at[cc, pl.ds(r0, ROWS_PER_TILE)])


_sc_agg_cnt = pl.kernel(
    functools.partial(_sc_body, True),
    out_type=(jax.ShapeDtypeStruct((NC, N_PAD, D), jnp.float32),
              jax.ShapeDtypeStruct((NC, N_PAD, 16), jnp.float32)),
    mesh=_mesh,
    scratch_types=[
        pltpu.VMEM((NCH, CH), jnp.int32),
        pltpu.VMEM((NCH, CH), jnp.int32),
        pltpu.VMEM((CH, D), jnp.float32),
        pltpu.VMEM((CH, 16), jnp.float32),
        pltpu.VMEM_SHARED((N_PAD, D), jnp.float32),
        pltpu.VMEM_SHARED((N_PAD, 16), jnp.float32),
        pltpu.SemaphoreType.DMA,
    ],
)

_sc_agg = pl.kernel(
    functools.partial(_sc_body, False),
    out_type=jax.ShapeDtypeStruct((NC, N_PAD, D), jnp.float32),
    mesh=_mesh,
    scratch_types=[
        pltpu.VMEM((NCH, CH), jnp.int32),
        pltpu.VMEM((NCH, CH), jnp.int32),
        pltpu.VMEM((CH, D), jnp.float32),
        pltpu.VMEM_SHARED((N_PAD, D), jnp.float32),
        pltpu.SemaphoreType.DMA,
    ],
)


BM = 1024
GRID = N_PAD // BM


def _tc_pre_body(x_ref, wl_ref, wr_ref, b_ref, g_ref, r_ref):
    h = x_ref[...]
    g_ref[...] = jnp.dot(h, wl_ref[...], preferred_element_type=jnp.float32)
    r_ref[...] = (jnp.dot(h, wr_ref[...], preferred_element_type=jnp.float32)
                  + b_ref[...])


def _tc_mid_body(s_ref, cnt_ref, rp_ref, wl_ref, wr_ref, b_ref, g_ref, r_ref):
    s = s_ref[0] + s_ref[1]
    c = cnt_ref[0] + cnt_ref[1]
    inv = 1.0 / jnp.maximum(c, 1.0)
    h = jnp.maximum(s * inv[:, None] + rp_ref[...], 0.0)
    g_ref[...] = jnp.dot(h, wl_ref[...], preferred_element_type=jnp.float32)
    r_ref[...] = (jnp.dot(h, wr_ref[...], preferred_element_type=jnp.float32)
                  + b_ref[...])


def _tc_fin_body(s_ref, cnt_ref, rp_ref, wlin_ref, blin_ref, o_ref):
    s = s_ref[0] + s_ref[1]
    c = cnt_ref[0] + cnt_ref[1]
    inv = 1.0 / jnp.maximum(c, 1.0)
    h = jnp.maximum(s * inv[:, None] + rp_ref[...], 0.0)
    o_ref[...] = (jnp.dot(h, wlin_ref[...], preferred_element_type=jnp.float32)
                  + blin_ref[...])


_w_spec = pl.BlockSpec((D, D), lambda i: (0, 0))
_b_spec = pl.BlockSpec((1, D), lambda i: (0, 0))
_row_spec = pl.BlockSpec((BM, D), lambda i: (i, 0))
_s_spec = pl.BlockSpec((NC, BM, D), lambda i: (0, i, 0))
_c_spec = pl.BlockSpec((NC, BM), lambda i: (0, i))

_tc_pre = pl.pallas_call(
    _tc_pre_body,
    grid=(GRID,),
    in_specs=[_row_spec, _w_spec, _w_spec, _b_spec],
    out_specs=[_row_spec, _row_spec],
    out_shape=[jax.ShapeDtypeStruct((N_PAD, D), jnp.float32)] * 2,
)

_tc_mid = pl.pallas_call(
    _tc_mid_body,
    grid=(GRID,),
    in_specs=[_s_spec, _c_spec, _row_spec, _w_spec, _w_spec, _b_spec],
    out_specs=[_row_spec, _row_spec],
    out_shape=[jax.ShapeDtypeStruct((N_PAD, D), jnp.float32)] * 2,
)

_tc_fin = pl.pallas_call(
    _tc_fin_body,
    grid=(GRID,),
    in_specs=[_s_spec, _c_spec, _row_spec, _w_spec, _b_spec],
    out_specs=_row_spec,
    out_shape=jax.ShapeDtypeStruct((N_PAD, D), jnp.float32),
)


def kernel(x, edge_index, Wl0, Wr0, b0, Wl1, Wr1, b1, Wl2, Wr2, b2,
           Wlin, blin):
    x_p = jnp.pad(x, ((0, N_PAD - N), (0, 0)))
    src = jnp.concatenate(
        [edge_index[0], jnp.zeros((E_PAD - E,), jnp.int32)])
    dst = jnp.concatenate(
        [edge_index[1], jnp.full((E_PAD - E,), N_PAD - 1, jnp.int32)])
    src_r = src.reshape(W, NCH, CH)
    dst_r = dst.reshape(W, NCH, CH)

    zeros = jnp.zeros((N_PAD, D), jnp.float32)
    zeros16 = jnp.zeros((N_PAD, 16), jnp.float32)
    ones = jnp.ones((CH, 16), jnp.float32)

    g0, r0 = _tc_pre(x_p, Wl0, Wr0, b0.reshape(1, D))
    s0, cnt16 = _sc_agg_cnt(g0, src_r, dst_r, zeros, zeros16, ones)
    cnt = cnt16[:, :, 0]
    g1, r1 = _tc_mid(s0, cnt, r0, Wl1, Wr1, b1.reshape(1, D))
    s1 = _sc_agg(g1, src_r, dst_r, zeros)
    g2, r2 = _tc_mid(s1, cnt, r1, Wl2, Wr2, b2.reshape(1, D))
    s2 = _sc_agg(g2, src_r, dst_r, zeros)
    out = _tc_fin(s2, cnt, r2, Wlin, blin.reshape(1, D))
    return out[:N]


# speedup vs baseline: 11.1338x; 11.1338x over previous
import functools

import jax
import jax.numpy as jnp
from jax import lax
from jax.experimental import pallas as pl
from jax.experimental.pallas import tpu as pltpu
from jax.experimental.pallas import tpu_sc as plsc

N = 10000
D = 128
E = 320000

N_PAD = 10240
NC, NS = 2, 16
W = NC * NS
CH = 128
EPW = 10240
NCH = EPW // CH
E_PAD = W * EPW
RPT = N_PAD // NS
CHA = 32
NBUF = 8
NCHA = EPW // CHA

_mesh = plsc.VectorSubcoreMesh(core_axis_name="c", subcore_axis_name="s",
                               num_cores=NC, num_subcores=NS)


def _sc_agg_body(*refs):
    (g_hbm, pk_hbm, z_hbm,
     out_hbm,
     pk_v, acc_sh) = refs[:6]
    srcs = refs[6:6 + NBUF]
    dsts = refs[6 + NBUF:6 + 2 * NBUF]
    rows = refs[6 + 2 * NBUF:6 + 3 * NBUF]
    sems = refs[6 + 3 * NBUF:6 + 4 * NBUF]

    cc = lax.axis_index("c")
    ss = lax.axis_index("s")
    wid = cc * NS + ss
    r0 = ss * RPT

    pltpu.sync_copy(z_hbm, acc_sh.at[pl.ds(r0, RPT)])
    pltpu.sync_copy(pk_hbm.at[wid], pk_v)

    bufs = tuple(zip(srcs, dsts, rows, sems))

    def unpack(c, src_c, dst_c):
        for k in range(CHA // 16):
            v = pk_v[pl.ds(c * CHA + k * 16, 16)]
            src_c[pl.ds(k * 16, 16)] = v & 16383
            dst_c[0, pl.ds(k * 16, 16)] = v >> 14

    for b, (src_c, dst_c, rows_v, sem) in enumerate(bufs):
        unpack(b, src_c, dst_c)
        pltpu.async_copy(g_hbm.at[src_c], rows_v, sem)
    plsc.subcore_barrier()

    def chunk(j, carry):
        for b, (src_c, dst_c, rows_v, sem) in enumerate(bufs):
            c = NBUF * j + b
            pltpu.make_async_copy(g_hbm.at[src_c], rows_v, sem).wait()
            pltpu.sync_copy(rows_v, acc_sh.at[dst_c.at[0]], add=True)
            nxt = c + NBUF

            @pl.when(nxt < NCHA)
            def _():
                unpack(nxt, src_c, dst_c)
                pltpu.async_copy(g_hbm.at[src_c], rows_v, sem)
        return carry

    lax.fori_loop(0, NCHA // NBUF, chunk, 0)
    plsc.subcore_barrier()

    pltpu.sync_copy(acc_sh.at[pl.ds(r0, RPT)],
                    out_hbm.at[cc, pl.ds(r0, RPT)])


def _sc_cnt_body(*refs):
    (pk_hbm, z_hbm, ones_hbm,
     cnt_hbm,
     pk_v, dst_v, ones_v, cnt_sh) = refs

    cc = lax.axis_index("c")
    ss = lax.axis_index("s")
    wid = cc * NS + ss
    r0 = ss * RPT

    pltpu.sync_copy(z_hbm, cnt_sh.at[pl.ds(r0, RPT)])
    pltpu.sync_copy(pk_hbm.at[wid], pk_v)
    pltpu.sync_copy(ones_hbm, ones_v)

    def unpack(i, carry):
        for k in range(CH // 16):
            v = pk_v[pl.ds(i * CH + k * 16, 16)]
            dst_v[i, pl.ds(k * 16, 16)] = v >> 14
        return carry

    lax.fori_loop(0, NCH, unpack, 0)
    plsc.subcore_barrier()

    def chunk(i, carry):
        pltpu.sync_copy(ones_v, cnt_sh.at[dst_v.at[i]], add=True)
        return carry

    lax.fori_loop(0, NCH, chunk, 0)
    plsc.subcore_barrier()
    pltpu.sync_copy(cnt_sh.at[pl.ds(r0, RPT)],
                    cnt_hbm.at[cc, pl.ds(r0, RPT)])


_sc_cnt = pl.kernel(
    _sc_cnt_body,
    out_type=jax.ShapeDtypeStruct((NC, N_PAD, D), jnp.float32),
    mesh=_mesh,
    scratch_types=[
        pltpu.VMEM((EPW,), jnp.int32),
        pltpu.VMEM((NCH, CH), jnp.int32),
        pltpu.VMEM((CH, D), jnp.float32),
        pltpu.VMEM_SHARED((N_PAD, D), jnp.float32),
    ],
)

_sc_agg = pl.kernel(
    _sc_agg_body,
    out_type=jax.ShapeDtypeStruct((NC, N_PAD, D), jnp.float32),
    mesh=_mesh,
    scratch_types=(
        [pltpu.VMEM((EPW,), jnp.int32),
         pltpu.VMEM_SHARED((N_PAD, D), jnp.float32)]
        + [pltpu.VMEM((CHA,), jnp.int32)] * NBUF
        + [pltpu.VMEM((1, CHA), jnp.int32)] * NBUF
        + [pltpu.VMEM((CHA, D), jnp.float32)] * NBUF
        + [pltpu.SemaphoreType.DMA] * NBUF
    ),
)


BM = 1024
GRID = N_PAD // BM


def _tc_pre_body(x_ref, wl_ref, wr_ref, b_ref, g_ref, r_ref):
    h = x_ref[...]
    g_ref[...] = jnp.dot(h, wl_ref[...], preferred_element_type=jnp.float32)
    r_ref[...] = (jnp.dot(h, wr_ref[...], preferred_element_type=jnp.float32)
                  + b_ref[...])


def _tc_mid_body(s_ref, cnt_ref, rp_ref, wl_ref, wr_ref, b_ref, g_ref, r_ref):
    s = s_ref[0] + s_ref[1]
    c = cnt_ref[0] + cnt_ref[1]
    inv = 1.0 / jnp.maximum(c, 1.0)
    h = jnp.maximum(s * inv[:, None] + rp_ref[...], 0.0)
    g_ref[...] = jnp.dot(h, wl_ref[...], preferred_element_type=jnp.float32)
    r_ref[...] = (jnp.dot(h, wr_ref[...], preferred_element_type=jnp.float32)
                  + b_ref[...])


def _tc_fin_body(s_ref, cnt_ref, rp_ref, wlin_ref, blin_ref, o_ref):
    s = s_ref[0] + s_ref[1]
    c = cnt_ref[0] + cnt_ref[1]
    inv = 1.0 / jnp.maximum(c, 1.0)
    h = jnp.maximum(s * inv[:, None] + rp_ref[...], 0.0)
    o_ref[...] = (jnp.dot(h, wlin_ref[...], preferred_element_type=jnp.float32)
                  + blin_ref[...])


_w_spec = pl.BlockSpec((D, D), lambda i: (0, 0))
_b_spec = pl.BlockSpec((1, D), lambda i: (0, 0))
_row_spec = pl.BlockSpec((BM, D), lambda i: (i, 0))
_s_spec = pl.BlockSpec((NC, BM, D), lambda i: (0, i, 0))
_c_spec = pl.BlockSpec((NC, BM), lambda i: (0, i))

_tc_pre = pl.pallas_call(
    _tc_pre_body,
    grid=(GRID,),
    in_specs=[_row_spec, _w_spec, _w_spec, _b_spec],
    out_specs=[_row_spec, _row_spec],
    out_shape=[jax.ShapeDtypeStruct((N_PAD, D), jnp.float32)] * 2,
)

_tc_mid = pl.pallas_call(
    _tc_mid_body,
    grid=(GRID,),
    in_specs=[_s_spec, _c_spec, _row_spec, _w_spec, _w_spec, _b_spec],
    out_specs=[_row_spec, _row_spec],
    out_shape=[jax.ShapeDtypeStruct((N_PAD, D), jnp.float32)] * 2,
)

_tc_fin = pl.pallas_call(
    _tc_fin_body,
    grid=(GRID,),
    in_specs=[_s_spec, _c_spec, _row_spec, _w_spec, _b_spec],
    out_specs=_row_spec,
    out_shape=jax.ShapeDtypeStruct((N_PAD, D), jnp.float32),
)


def kernel(x, edge_index, Wl0, Wr0, b0, Wl1, Wr1, b1, Wl2, Wr2, b2,
           Wlin, blin):
    x_p = jnp.pad(x, ((0, N_PAD - N), (0, 0)))
    fill = N + jnp.arange(E_PAD - E, dtype=jnp.int32) % (N_PAD - N)
    src = jnp.concatenate([edge_index[0], fill])
    dst = jnp.concatenate([edge_index[1], fill])
    pk = (src | (dst << 14)).reshape(W, EPW)

    zeros = jnp.zeros((RPT, D), jnp.float32)
    ones = jnp.ones((CH, D), jnp.float32)

    g0, r0 = _tc_pre(x_p, Wl0, Wr0, b0.reshape(1, D))
    cnt128 = _sc_cnt(pk, zeros, ones)
    s0 = _sc_agg(g0, pk, zeros)
    cnt = cnt128[:, :, 0]
    g1, r1 = _tc_mid(s0, cnt, r0, Wl1, Wr1, b1.reshape(1, D))
    s1 = _sc_agg(g1, pk, zeros)
    g2, r2 = _tc_mid(s1, cnt, r1, Wl2, Wr2, b2.reshape(1, D))
    s2 = _sc_agg(g2, pk, zeros)
    out = _tc_fin(s2, cnt, r2, Wlin, blin.reshape(1, D))
    return out[:N]


# speedup vs baseline: 11.1604x; 1.0024x over previous
import functools

import jax
import jax.numpy as jnp
from jax import lax
from jax.experimental import pallas as pl
from jax.experimental.pallas import tpu as pltpu
from jax.experimental.pallas import tpu_sc as plsc

N = 10000
D = 128
E = 320000

N_PAD = 10240
NC, NS = 2, 16
W = NC * NS
CH = 128
EPW = 10240
NCH = EPW // CH
E_PAD = W * EPW
RPT = N_PAD // NS
CHA = 64
NBUF = 4
NCHA = EPW // CHA

_mesh = plsc.VectorSubcoreMesh(core_axis_name="c", subcore_axis_name="s",
                               num_cores=NC, num_subcores=NS)


def _sc_agg_body(*refs):
    (g_hbm, pk_hbm, z_hbm,
     out_hbm,
     pk_v, acc_sh) = refs[:6]
    srcs = refs[6:6 + NBUF]
    dsts = refs[6 + NBUF:6 + 2 * NBUF]
    rows = refs[6 + 2 * NBUF:6 + 3 * NBUF]
    sems = refs[6 + 3 * NBUF:6 + 4 * NBUF]

    cc = lax.axis_index("c")
    ss = lax.axis_index("s")
    wid = cc * NS + ss
    r0 = ss * RPT

    pltpu.sync_copy(z_hbm, acc_sh.at[pl.ds(r0, RPT)])
    pltpu.sync_copy(pk_hbm.at[wid], pk_v)

    bufs = tuple(zip(srcs, dsts, rows, sems))

    def unpack(c, src_c, dst_c):
        for k in range(CHA // 16):
            v = pk_v[pl.ds(c * CHA + k * 16, 16)]
            src_c[pl.ds(k * 16, 16)] = v & 16383
            dst_c[0, pl.ds(k * 16, 16)] = v >> 14

    for b, (src_c, dst_c, rows_v, sem) in enumerate(bufs):
        unpack(b, src_c, dst_c)
        pltpu.async_copy(g_hbm.at[src_c], rows_v, sem)
    plsc.subcore_barrier()

    def chunk(j, carry):
        for b, (src_c, dst_c, rows_v, sem) in enumerate(bufs):
            c = NBUF * j + b
            pltpu.make_async_copy(g_hbm.at[src_c], rows_v, sem).wait()
            pltpu.sync_copy(rows_v, acc_sh.at[dst_c.at[0]], add=True)
            nxt = c + NBUF

            @pl.when(nxt < NCHA)
            def _():
                unpack(nxt, src_c, dst_c)
                pltpu.async_copy(g_hbm.at[src_c], rows_v, sem)
        return carry

    lax.fori_loop(0, NCHA // NBUF, chunk, 0)
    plsc.subcore_barrier()

    pltpu.sync_copy(acc_sh.at[pl.ds(r0, RPT)],
                    out_hbm.at[cc, pl.ds(r0, RPT)])


def _sc_cnt_body(*refs):
    (pk_hbm, z_hbm, ones_hbm,
     cnt_hbm,
     pk_v, dst_v, ones_v, cnt_sh) = refs

    cc = lax.axis_index("c")
    ss = lax.axis_index("s")
    wid = cc * NS + ss
    r0 = ss * RPT

    pltpu.sync_copy(z_hbm, cnt_sh.at[pl.ds(r0, RPT)])
    pltpu.sync_copy(pk_hbm.at[wid], pk_v)
    pltpu.sync_copy(ones_hbm, ones_v)

    def unpack(i, carry):
        for k in range(CH // 16):
            v = pk_v[pl.ds(i * CH + k * 16, 16)]
            dst_v[i, pl.ds(k * 16, 16)] = v >> 14
        return carry

    lax.fori_loop(0, NCH, unpack, 0)
    plsc.subcore_barrier()

    def chunk(i, carry):
        pltpu.sync_copy(ones_v, cnt_sh.at[dst_v.at[i]], add=True)
        return carry

    lax.fori_loop(0, NCH, chunk, 0)
    plsc.subcore_barrier()
    pltpu.sync_copy(cnt_sh.at[pl.ds(r0, RPT)],
                    cnt_hbm.at[cc, pl.ds(r0, RPT)])


_sc_cnt = pl.kernel(
    _sc_cnt_body,
    out_type=jax.ShapeDtypeStruct((NC, N_PAD, D), jnp.float32),
    mesh=_mesh,
    scratch_types=[
        pltpu.VMEM((EPW,), jnp.int32),
        pltpu.VMEM((NCH, CH), jnp.int32),
        pltpu.VMEM((CH, D), jnp.float32),
        pltpu.VMEM_SHARED((N_PAD, D), jnp.float32),
    ],
)

_sc_agg = pl.kernel(
    _sc_agg_body,
    out_type=jax.ShapeDtypeStruct((NC, N_PAD, D), jnp.float32),
    mesh=_mesh,
    scratch_types=(
        [pltpu.VMEM((EPW,), jnp.int32),
         pltpu.VMEM_SHARED((N_PAD, D), jnp.float32)]
        + [pltpu.VMEM((CHA,), jnp.int32)] * NBUF
        + [pltpu.VMEM((1, CHA), jnp.int32)] * NBUF
        + [pltpu.VMEM((CHA, D), jnp.float32)] * NBUF
        + [pltpu.SemaphoreType.DMA] * NBUF
    ),
)


BM = 1024
GRID = N_PAD // BM


def _tc_pre_body(x_ref, wl_ref, wr_ref, b_ref, g_ref, r_ref):
    h = x_ref[...]
    g_ref[...] = jnp.dot(h, wl_ref[...], preferred_element_type=jnp.float32)
    r_ref[...] = (jnp.dot(h, wr_ref[...], preferred_element_type=jnp.float32)
                  + b_ref[...])


def _tc_mid_body(s_ref, cnt_ref, rp_ref, wl_ref, wr_ref, b_ref, g_ref, r_ref):
    s = s_ref[0] + s_ref[1]
    c = cnt_ref[0] + cnt_ref[1]
    inv = 1.0 / jnp.maximum(c, 1.0)
    h = jnp.maximum(s * inv[:, None] + rp_ref[...], 0.0)
    g_ref[...] = jnp.dot(h, wl_ref[...], preferred_element_type=jnp.float32)
    r_ref[...] = (jnp.dot(h, wr_ref[...], preferred_element_type=jnp.float32)
                  + b_ref[...])


def _tc_fin_body(s_ref, cnt_ref, rp_ref, wlin_ref, blin_ref, o_ref):
    s = s_ref[0] + s_ref[1]
    c = cnt_ref[0] + cnt_ref[1]
    inv = 1.0 / jnp.maximum(c, 1.0)
    h = jnp.maximum(s * inv[:, None] + rp_ref[...], 0.0)
    o_ref[...] = (jnp.dot(h, wlin_ref[...], preferred_element_type=jnp.float32)
                  + blin_ref[...])


_w_spec = pl.BlockSpec((D, D), lambda i: (0, 0))
_b_spec = pl.BlockSpec((1, D), lambda i: (0, 0))
_row_spec = pl.BlockSpec((BM, D), lambda i: (i, 0))
_s_spec = pl.BlockSpec((NC, BM, D), lambda i: (0, i, 0))
_c_spec = pl.BlockSpec((NC, BM), lambda i: (0, i))

_tc_pre = pl.pallas_call(
    _tc_pre_body,
    grid=(GRID,),
    in_specs=[_row_spec, _w_spec, _w_spec, _b_spec],
    out_specs=[_row_spec, _row_spec],
    out_shape=[jax.ShapeDtypeStruct((N_PAD, D), jnp.float32)] * 2,
)

_tc_mid = pl.pallas_call(
    _tc_mid_body,
    grid=(GRID,),
    in_specs=[_s_spec, _c_spec, _row_spec, _w_spec, _w_spec, _b_spec],
    out_specs=[_row_spec, _row_spec],
    out_shape=[jax.ShapeDtypeStruct((N_PAD, D), jnp.float32)] * 2,
)

_tc_fin = pl.pallas_call(
    _tc_fin_body,
    grid=(GRID,),
    in_specs=[_s_spec, _c_spec, _row_spec, _w_spec, _b_spec],
    out_specs=_row_spec,
    out_shape=jax.ShapeDtypeStruct((N_PAD, D), jnp.float32),
)


def kernel(x, edge_index, Wl0, Wr0, b0, Wl1, Wr1, b1, Wl2, Wr2, b2,
           Wlin, blin):
    x_p = jnp.pad(x, ((0, N_PAD - N), (0, 0)))
    fill = N + jnp.arange(E_PAD - E, dtype=jnp.int32) % (N_PAD - N)
    src = jnp.concatenate([edge_index[0], fill])
    dst = jnp.concatenate([edge_index[1], fill])
    pk = (src | (dst << 14)).reshape(W, EPW)

    zeros = jnp.zeros((RPT, D), jnp.float32)
    ones = jnp.ones((CH, D), jnp.float32)

    g0, r0 = _tc_pre(x_p, Wl0, Wr0, b0.reshape(1, D))
    cnt128 = _sc_cnt(pk, zeros, ones)
    s0 = _sc_agg(g0, pk, zeros)
    cnt = cnt128[:, :, 0]
    g1, r1 = _tc_mid(s0, cnt, r0, Wl1, Wr1, b1.reshape(1, D))
    s1 = _sc_agg(g1, pk, zeros)
    g2, r2 = _tc_mid(s1, cnt, r1, Wl2, Wr2, b2.reshape(1, D))
    s2 = _sc_agg(g2, pk, zeros)
    out = _tc_fin(s2, cnt, r2, Wlin, blin.reshape(1, D))
    return out[:N]
